# Initial kernel scaffold; baseline (speedup 1.0000x reference)
import functools

import jax
import jax.numpy as jnp
from jax import lax
from jax.experimental import pallas as pl
from jax.experimental.pallas import tpu as pltpu
from jax.experimental.pallas import tpu_sc as plsc

N = 10000
E = 160000
D = 256
H = 128
NC = 2
NS = 16
B = 128
TPW = 10240
EP = NS * TPW
NB = TPW // B
ACC_R = 10240
RPT = ACC_R // NS


def _sc_aggregate(gidx, sidx, flo, fhi, zrows, z16, ones16):
  mesh = plsc.VectorSubcoreMesh(core_axis_name="c", subcore_axis_name="s")

  @functools.partial(
      pl.kernel,
      out_type=[
          jax.ShapeDtypeStruct((NC, 2, ACC_R, H), jnp.float32),
          jax.ShapeDtypeStruct((NC, ACC_R, 16), jnp.float32),
      ],
      mesh=mesh,
      scratch_types=[
          pltpu.VMEM_SHARED((ACC_R, H), jnp.float32),
          pltpu.VMEM_SHARED((ACC_R, 16), jnp.float32),
          pltpu.VMEM((B,), jnp.int32),
          pltpu.VMEM((B,), jnp.int32),
          pltpu.VMEM((B, H), jnp.float32),
          pltpu.VMEM((B, H), jnp.float32),
          pltpu.VMEM((RPT, 16), jnp.float32),
          pltpu.VMEM((B, 16), jnp.float32),
          pltpu.SemaphoreType.DMA,
      ],
  )
  def body(gidx_h, sidx_h, flo_h, fhi_h, zrows_h, z16_h, ones_h,
           aggs_o, cnts_o, acc_sh, cnt_sh, idxg, idxs, rows, zbuf, z16v,
           ones_v, sem):
    c = lax.axis_index("c")
    s = lax.axis_index("s")
    rbase = s * RPT

    pltpu.sync_copy(zrows_h, zbuf)
    pltpu.sync_copy(z16_h, z16v)
    pltpu.sync_copy(ones_h, ones_v)
    pltpu.sync_copy(z16v, cnt_sh.at[pl.ds(rbase, RPT)])

    for h in range(2):
      fsrc = flo_h if h == 0 else fhi_h
      for j in range(RPT // B):
        pltpu.sync_copy(zbuf, acc_sh.at[pl.ds(rbase + j * B, B)])
      plsc.subcore_barrier()

      @pl.loop(0, NB)
      def batch_loop(g):
        off = s * TPW + g * B
        pltpu.sync_copy(gidx_h.at[c, pl.ds(off, B)], idxg)
        pltpu.sync_copy(sidx_h.at[c, pl.ds(off, B)], idxs)
        pltpu.async_copy(fsrc.at[idxg], rows, sem).wait()
        pltpu.sync_copy(rows, acc_sh.at[idxs], add=True)
        if h == 0:
          pltpu.sync_copy(ones_v, cnt_sh.at[idxs], add=True)

      plsc.subcore_barrier()
      pltpu.sync_copy(acc_sh.at[pl.ds(rbase, RPT)],
                      aggs_o.at[c, h, pl.ds(rbase, RPT)])

    pltpu.sync_copy(cnt_sh.at[pl.ds(rbase, RPT)],
                    cnts_o.at[c, pl.ds(rbase, RPT)])

  return body(gidx, sidx, flo, fhi, zrows, z16, ones16)


def _tc_combine(aggs, cnts, Wt, bstack):
  RB = 256
  grid = (ACC_R // RB,)

  def body(agg_ref, cnt_ref, wt_ref, b_ref, out_ref):
    cf = cnt_ref[0, :, 0:1]
    cb = cnt_ref[1, :, 0:1]
    acc = jnp.dot(agg_ref[0, 0], wt_ref[0:H],
                  preferred_element_type=jnp.float32)
    acc += jnp.dot(agg_ref[0, 1], wt_ref[H:2 * H],
                   preferred_element_type=jnp.float32)
    acc += jnp.dot(agg_ref[1, 0], wt_ref[2 * H:3 * H],
                   preferred_element_type=jnp.float32)
    acc += jnp.dot(agg_ref[1, 1], wt_ref[3 * H:4 * H],
                   preferred_element_type=jnp.float32)
    acc += cf * b_ref[0:1, :] + cb * b_ref[1:2, :]
    deg = cf + cb
    deg = jnp.where(deg == 0.0, 1.0, deg)
    out_ref[...] = jnp.maximum(acc / deg, 0.0)

  return pl.pallas_call(
      body,
      grid=grid,
      in_specs=[
          pl.BlockSpec((NC, 2, RB, H), lambda i: (0, 0, i, 0)),
          pl.BlockSpec((NC, RB, 16), lambda i: (0, i, 0)),
          pl.BlockSpec((4 * H, D), lambda i: (0, 0)),
          pl.BlockSpec((2, D), lambda i: (0, 0)),
      ],
      out_specs=pl.BlockSpec((RB, D), lambda i: (i, 0)),
      out_shape=jax.ShapeDtypeStruct((ACC_R, D), jnp.float32),
  )(aggs, cnts, Wt, bstack)


def kernel(feat, edge_index, Wf, bf, Wb, bb):
  src = edge_index[0]
  dst = edge_index[1]
  npad = EP - E
  pad0 = jnp.zeros((npad,), jnp.int32)
  padN = jnp.full((npad,), N, jnp.int32)
  gidx = jnp.stack([jnp.concatenate([src, pad0]),
                    jnp.concatenate([dst, pad0])])
  sidx = jnp.stack([jnp.concatenate([dst, padN]),
                    jnp.concatenate([src, padN])])
  flo = feat[:, :H]
  fhi = feat[:, H:]
  zrows = jnp.zeros((B, H), jnp.float32)
  z16 = jnp.zeros((RPT, 16), jnp.float32)
  ones16 = jnp.ones((B, 16), jnp.float32)

  aggs, cnts = _sc_aggregate(gidx, sidx, flo, fhi, zrows, z16, ones16)

  Wt = jnp.concatenate([Wf.T, Wb.T], axis=0)
  bstack = jnp.stack([bf, bb])
  out = _tc_combine(aggs, cnts, Wt, bstack)
  return out[:N]


# speedup vs baseline: 1.8029x; 1.8029x over previous
import functools

import jax
import jax.numpy as jnp
from jax import lax
from jax.experimental import pallas as pl
from jax.experimental.pallas import tpu as pltpu
from jax.experimental.pallas import tpu_sc as plsc

N = 10000
E = 160000
D = 256
H = 128
NC = 2
NS = 16
B = 128
TPW = 10240
EP = NS * TPW
NB = TPW // B
ACC_R = 10240
RPT = ACC_R // NS


def _sc_aggregate(gidx, sidx, flo, fhi, zrows, ones16):
  mesh = plsc.VectorSubcoreMesh(core_axis_name="c", subcore_axis_name="s")

  @functools.partial(
      pl.kernel,
      out_type=[
          jax.ShapeDtypeStruct((NC, 2, ACC_R, H), jnp.float32),
          jax.ShapeDtypeStruct((NC, ACC_R, 16), jnp.float32),
      ],
      mesh=mesh,
      compiler_params=pltpu.CompilerParams(use_tc_tiling_on_sc=False),
      scratch_types=[
          pltpu.VMEM_SHARED((ACC_R, H), jnp.float32),
          pltpu.VMEM_SHARED((ACC_R, 16), jnp.float32),
          pltpu.VMEM((B,), jnp.int32),
          pltpu.VMEM((B,), jnp.int32),
          pltpu.VMEM((B, H), jnp.float32),
          pltpu.VMEM((B, 16), jnp.float32),
          pltpu.SemaphoreType.DMA,
      ],
  )
  def body(gidx_h, sidx_h, flo_h, fhi_h, zrows_h, ones_h,
           aggs_o, cnts_o, acc_sh, cnt_sh, idxg, idxs, rows,
           ones_v, sem):
    c = lax.axis_index("c")
    s = lax.axis_index("s")
    rbase = s * RPT

    pltpu.sync_copy(ones_h, ones_v)

    for h in range(2):
      fsrc = flo_h if h == 0 else fhi_h
      pltpu.sync_copy(zrows_h, rows)
      for j in range(RPT // B):
        pltpu.sync_copy(rows, acc_sh.at[pl.ds(rbase + j * B, B)])
        if h == 0:
          pltpu.sync_copy(rows.at[pl.ds(0, B), pl.ds(0, 16)],
                          cnt_sh.at[pl.ds(rbase + j * B, B)])
      plsc.subcore_barrier()

      @pl.loop(0, NB)
      def batch_loop(g):
        off = s * TPW + g * B
        pltpu.sync_copy(gidx_h.at[c, pl.ds(off, B)], idxg)
        pltpu.sync_copy(sidx_h.at[c, pl.ds(off, B)], idxs)
        pltpu.async_copy(fsrc.at[idxg], rows, sem).wait()
        pltpu.sync_copy(rows, acc_sh.at[idxs], add=True)
        if h == 0:
          pltpu.sync_copy(ones_v, cnt_sh.at[idxs], add=True)

      plsc.subcore_barrier()
      pltpu.sync_copy(acc_sh.at[pl.ds(rbase, RPT)],
                      aggs_o.at[c, h, pl.ds(rbase, RPT)])

    pltpu.sync_copy(cnt_sh.at[pl.ds(rbase, RPT)],
                    cnts_o.at[c, pl.ds(rbase, RPT)])

  return body(gidx, sidx, flo, fhi, zrows, ones16)


def _tc_combine(aggs, cnts, Wt, bstack):
  RB = 256
  grid = (ACC_R // RB,)

  def body(agg_ref, cnt_ref, wt_ref, b_ref, out_ref):
    cf = cnt_ref[0, :, 0:1]
    cb = cnt_ref[1, :, 0:1]
    acc = jnp.dot(agg_ref[0, 0], wt_ref[0:H],
                  preferred_element_type=jnp.float32)
    acc += jnp.dot(agg_ref[0, 1], wt_ref[H:2 * H],
                   preferred_element_type=jnp.float32)
    acc += jnp.dot(agg_ref[1, 0], wt_ref[2 * H:3 * H],
                   preferred_element_type=jnp.float32)
    acc += jnp.dot(agg_ref[1, 1], wt_ref[3 * H:4 * H],
                   preferred_element_type=jnp.float32)
    acc += cf * b_ref[0:1, :] + cb * b_ref[1:2, :]
    deg = cf + cb
    deg = jnp.where(deg == 0.0, 1.0, deg)
    out_ref[...] = jnp.maximum(acc / deg, 0.0)

  return pl.pallas_call(
      body,
      grid=grid,
      in_specs=[
          pl.BlockSpec((NC, 2, RB, H), lambda i: (0, 0, i, 0)),
          pl.BlockSpec((NC, RB, 16), lambda i: (0, i, 0)),
          pl.BlockSpec((4 * H, D), lambda i: (0, 0)),
          pl.BlockSpec((2, D), lambda i: (0, 0)),
      ],
      out_specs=pl.BlockSpec((RB, D), lambda i: (i, 0)),
      out_shape=jax.ShapeDtypeStruct((ACC_R, D), jnp.float32),
  )(aggs, cnts, Wt, bstack)


def kernel(feat, edge_index, Wf, bf, Wb, bb):
  src = edge_index[0]
  dst = edge_index[1]
  npad = EP - E
  pad0 = jnp.zeros((npad,), jnp.int32)
  padN = jnp.full((npad,), N, jnp.int32)
  gidx = jnp.stack([jnp.concatenate([src, pad0]),
                    jnp.concatenate([dst, pad0])])
  sidx = jnp.stack([jnp.concatenate([dst, padN]),
                    jnp.concatenate([src, padN])])
  flo = feat[:, :H]
  fhi = feat[:, H:]
  zrows = jnp.zeros((B, H), jnp.float32)
  ones16 = jnp.ones((B, 16), jnp.float32)

  aggs, cnts = _sc_aggregate(gidx, sidx, flo, fhi, zrows, ones16)

  Wt = jnp.concatenate([Wf.T, Wb.T], axis=0)
  bstack = jnp.stack([bf, bb])
  out = _tc_combine(aggs, cnts, Wt, bstack)
  return out[:N]


# speedup vs baseline: 2.0152x; 1.1177x over previous
import functools

import jax
import jax.numpy as jnp
from jax import lax
from jax.experimental import pallas as pl
from jax.experimental.pallas import tpu as pltpu
from jax.experimental.pallas import tpu_sc as plsc

N = 10000
E = 160000
D = 256
H = 128
NC = 2
NS = 16
B = 128
TPW = 10240
EP = NS * TPW
NB = TPW // B
ACC_R = 10240
RPT = ACC_R // NS


def _sc_aggregate(gidx, sidx, flo, fhi, zrows, ones16):
  mesh = plsc.VectorSubcoreMesh(core_axis_name="c", subcore_axis_name="s")

  @functools.partial(
      pl.kernel,
      out_type=[
          jax.ShapeDtypeStruct((NC, 2, ACC_R, H), jnp.float32),
          jax.ShapeDtypeStruct((NC, ACC_R, 16), jnp.float32),
      ],
      mesh=mesh,
      compiler_params=pltpu.CompilerParams(use_tc_tiling_on_sc=False),
      scratch_types=[
          pltpu.VMEM_SHARED((ACC_R, H), jnp.float32),
          pltpu.VMEM_SHARED((ACC_R, 16), jnp.float32),
          pltpu.VMEM((B,), jnp.int32),
          pltpu.VMEM((B,), jnp.int32),
          pltpu.VMEM((B,), jnp.int32),
          pltpu.VMEM((B,), jnp.int32),
          pltpu.VMEM((B, H), jnp.float32),
          pltpu.VMEM((B, H), jnp.float32),
          pltpu.VMEM((B, 16), jnp.float32),
          pltpu.SemaphoreType.DMA,
          pltpu.SemaphoreType.DMA,
      ],
  )
  def body(gidx_h, sidx_h, flo_h, fhi_h, zrows_h, ones_h,
           aggs_o, cnts_o, acc_sh, cnt_sh, idxg0, idxs0, idxg1, idxs1,
           rows0, rows1, ones_v, sem0, sem1):
    c = lax.axis_index("c")
    s = lax.axis_index("s")
    rbase = s * RPT

    pltpu.sync_copy(ones_h, ones_v)

    for h in range(2):
      fsrc = flo_h if h == 0 else fhi_h
      pltpu.sync_copy(zrows_h, rows0)
      for j in range(RPT // B):
        pltpu.sync_copy(rows0, acc_sh.at[pl.ds(rbase + j * B, B)])
        if h == 0:
          pltpu.sync_copy(rows0.at[pl.ds(0, B), pl.ds(0, 16)],
                          cnt_sh.at[pl.ds(rbase + j * B, B)])
      plsc.subcore_barrier()

      @pl.loop(0, NB, step=2)
      def batch_loop(g):
        off0 = s * TPW + g * B
        pltpu.sync_copy(gidx_h.at[c, pl.ds(off0, B)], idxg0)
        cp0 = pltpu.async_copy(fsrc.at[idxg0], rows0, sem0)
        pltpu.sync_copy(gidx_h.at[c, pl.ds(off0 + B, B)], idxg1)
        cp1 = pltpu.async_copy(fsrc.at[idxg1], rows1, sem1)
        pltpu.sync_copy(sidx_h.at[c, pl.ds(off0, B)], idxs0)
        pltpu.sync_copy(sidx_h.at[c, pl.ds(off0 + B, B)], idxs1)
        cp0.wait()
        pltpu.sync_copy(rows0, acc_sh.at[idxs0], add=True)
        if h == 0:
          pltpu.sync_copy(ones_v, cnt_sh.at[idxs0], add=True)
        cp1.wait()
        pltpu.sync_copy(rows1, acc_sh.at[idxs1], add=True)
        if h == 0:
          pltpu.sync_copy(ones_v, cnt_sh.at[idxs1], add=True)

      plsc.subcore_barrier()
      pltpu.sync_copy(acc_sh.at[pl.ds(rbase, RPT)],
                      aggs_o.at[c, h, pl.ds(rbase, RPT)])

    pltpu.sync_copy(cnt_sh.at[pl.ds(rbase, RPT)],
                    cnts_o.at[c, pl.ds(rbase, RPT)])

  return body(gidx, sidx, flo, fhi, zrows, ones16)


def _tc_combine(aggs, cnts, Wt, bstack):
  RB = 256
  grid = (ACC_R // RB,)

  def body(agg_ref, cnt_ref, wt_ref, b_ref, out_ref):
    cf = cnt_ref[0, :, 0:1]
    cb = cnt_ref[1, :, 0:1]
    acc = jnp.dot(agg_ref[0, 0], wt_ref[0:H],
                  preferred_element_type=jnp.float32)
    acc += jnp.dot(agg_ref[0, 1], wt_ref[H:2 * H],
                   preferred_element_type=jnp.float32)
    acc += jnp.dot(agg_ref[1, 0], wt_ref[2 * H:3 * H],
                   preferred_element_type=jnp.float32)
    acc += jnp.dot(agg_ref[1, 1], wt_ref[3 * H:4 * H],
                   preferred_element_type=jnp.float32)
    acc += cf * b_ref[0:1, :] + cb * b_ref[1:2, :]
    deg = cf + cb
    deg = jnp.where(deg == 0.0, 1.0, deg)
    out_ref[...] = jnp.maximum(acc / deg, 0.0)

  return pl.pallas_call(
      body,
      grid=grid,
      in_specs=[
          pl.BlockSpec((NC, 2, RB, H), lambda i: (0, 0, i, 0)),
          pl.BlockSpec((NC, RB, 16), lambda i: (0, i, 0)),
          pl.BlockSpec((4 * H, D), lambda i: (0, 0)),
          pl.BlockSpec((2, D), lambda i: (0, 0)),
      ],
      out_specs=pl.BlockSpec((RB, D), lambda i: (i, 0)),
      out_shape=jax.ShapeDtypeStruct((ACC_R, D), jnp.float32),
  )(aggs, cnts, Wt, bstack)


def kernel(feat, edge_index, Wf, bf, Wb, bb):
  src = edge_index[0]
  dst = edge_index[1]
  npad = EP - E
  pad0 = jnp.zeros((npad,), jnp.int32)
  padN = jnp.full((npad,), N, jnp.int32)
  gidx = jnp.stack([jnp.concatenate([src, pad0]),
                    jnp.concatenate([dst, pad0])])
  sidx = jnp.stack([jnp.concatenate([dst, padN]),
                    jnp.concatenate([src, padN])])
  flo = feat[:, :H]
  fhi = feat[:, H:]
  zrows = jnp.zeros((B, H), jnp.float32)
  ones16 = jnp.ones((B, 16), jnp.float32)

  aggs, cnts = _sc_aggregate(gidx, sidx, flo, fhi, zrows, ones16)

  Wt = jnp.concatenate([Wf.T, Wb.T], axis=0)
  bstack = jnp.stack([bf, bb])
  out = _tc_combine(aggs, cnts, Wt, bstack)
  return out[:N]
